# baseline (speedup 1.0000x reference)
import functools
import jax
import jax.numpy as jnp
from jax import lax
from jax.experimental import pallas as pl
from jax.experimental.pallas import tpu as pltpu
from jax.experimental.pallas import tpu_sc as plsc

D_MODEL = 64
SCALE = 8.0

NC = 2
NS = 16
NW = NC * NS

B_TOTAL = 4096 * 200
B_PER_W = B_TOTAL // NW
CHUNK = 128
N_CHUNKS = B_PER_W // CHUNK


def _emb_body(idx_hbm, table_hbm, out_hbm, idx_v, rows_v, gsem):
    wid = lax.axis_index("s") * NC + lax.axis_index("c")
    base = wid * B_PER_W

    pltpu.sync_copy(idx_hbm.at[wid], idx_v)

    def chunk_step(j, _):
        pltpu.async_copy(table_hbm.at[idx_v.at[j]], rows_v, gsem).wait()

        def scale_row(r, _):
            for c in range(D_MODEL // 16):
                sl = pl.ds(c * 16, 16)
                rows_v[r, sl] = rows_v[r, sl] * SCALE
            return ()

        lax.fori_loop(0, CHUNK, scale_row, ())

        pltpu.sync_copy(rows_v, out_hbm.at[pl.ds(base + j * CHUNK, CHUNK)])
        return ()

    lax.fori_loop(0, N_CHUNKS, chunk_step, ())


@functools.partial(jax.jit, static_argnums=())
def _emb_call(idx, weight):
    mesh = plsc.VectorSubcoreMesh(
        core_axis_name="c", subcore_axis_name="s", num_cores=NC, num_subcores=NS
    )
    fn = pl.kernel(
        _emb_body,
        out_type=jax.ShapeDtypeStruct((B_TOTAL, D_MODEL), jnp.float32),
        mesh=mesh,
        scratch_types=[
            pltpu.VMEM((N_CHUNKS, CHUNK), jnp.int32),
            pltpu.VMEM((CHUNK, D_MODEL), jnp.float32),
            pltpu.SemaphoreType.DMA,
        ],
        compiler_params=pltpu.CompilerParams(use_tc_tiling_on_sc=False),
    )
    return fn(idx, weight)


def kernel(token_embedding, weight):
    idx = token_embedding.reshape(NW, N_CHUNKS, CHUNK)
    out = _emb_call(idx, weight)
    return out.reshape(4096, 200, D_MODEL)


# speedup vs baseline: 1.2078x; 1.2078x over previous
import functools
import jax
import jax.numpy as jnp
from jax import lax
from jax.experimental import pallas as pl
from jax.experimental.pallas import tpu as pltpu
from jax.experimental.pallas import tpu_sc as plsc

D_MODEL = 64
SCALE = 8.0

NC = 2
NS = 16
NW = NC * NS

B_TOTAL = 4096 * 200
B_PER_W = B_TOTAL // NW
CHUNK = 128
N_CHUNKS = B_PER_W // CHUNK

NI = 4
NO = 4


def _emb_body(idx_hbm, table_hbm, out_hbm, idx_v, in_v, out_v, gsem, osem):
    wid = lax.axis_index("s") * NC + lax.axis_index("c")
    base = wid * B_PER_W

    pltpu.sync_copy(idx_hbm.at[wid], idx_v)

    def gather(j, bi):
        return pltpu.async_copy(
            table_hbm.at[idx_v.at[j]], in_v.at[bi], gsem.at[bi]
        )

    def writeback(j, bo):
        return pltpu.async_copy(
            out_v.at[bo], out_hbm.at[pl.ds(base + j * CHUNK, CHUNK)], osem.at[bo]
        )

    for b in range(NI):
        gather(b, b)

    def step(j, bi, bo, first_round):
        pltpu.make_async_copy(
            table_hbm.at[idx_v.at[j]], in_v.at[bi], gsem.at[bi]
        ).wait()
        if not first_round:
            pltpu.make_async_copy(
                out_v.at[bo], out_hbm.at[pl.ds(base, CHUNK)], osem.at[bo]
            ).wait()

        def scale_rows(r, _):
            for rr in range(2):
                for c in range(D_MODEL // 16):
                    sl = pl.ds(c * 16, 16)
                    out_v[bo, r * 2 + rr, sl] = in_v[bi, r * 2 + rr, sl] * SCALE
            return ()

        lax.fori_loop(0, CHUNK // 2, scale_rows, ())

        writeback(j, bo)

    for b in range(NO):
        step(b, b % NI, b, True)
        gather(b + NI, b % NI)

    period = NI * NO // _gcd(NI, NO)

    def steady(g, _):
        j0 = NO + g * period
        for p in range(period):
            j = j0 + p
            bi = (NO + p) % NI
            bo = (NO + p) % NO
            step(j, bi, bo, False)

            @pl.when(j + NI < N_CHUNKS)
            def _():
                gather(j + NI, bi)

        return ()

    n_steady = (N_CHUNKS - NO) // period
    lax.fori_loop(0, n_steady, steady, ())

    for j in range(NO + n_steady * period, N_CHUNKS):
        step(j, j % NI, j % NO, False)

    for j in range(N_CHUNKS - NO, N_CHUNKS):
        pltpu.make_async_copy(
            out_v.at[j % NO], out_hbm.at[pl.ds(base, CHUNK)], osem.at[j % NO]
        ).wait()


def _gcd(a, b):
    while b:
        a, b = b, a % b
    return a


@jax.jit
def _emb_call(idx, weight):
    mesh = plsc.VectorSubcoreMesh(
        core_axis_name="c", subcore_axis_name="s", num_cores=NC, num_subcores=NS
    )
    fn = pl.kernel(
        _emb_body,
        out_type=jax.ShapeDtypeStruct((B_TOTAL, D_MODEL), jnp.float32),
        mesh=mesh,
        scratch_types=[
            pltpu.VMEM((N_CHUNKS, CHUNK), jnp.int32),
            pltpu.VMEM((NI, CHUNK, D_MODEL), jnp.float32),
            pltpu.VMEM((NO, CHUNK, D_MODEL), jnp.float32),
            pltpu.SemaphoreType.DMA((NI,)),
            pltpu.SemaphoreType.DMA((NO,)),
        ],
        compiler_params=pltpu.CompilerParams(use_tc_tiling_on_sc=False),
    )
    return fn(idx, weight)


def kernel(token_embedding, weight):
    idx = token_embedding.reshape(NW, N_CHUNKS, CHUNK)
    out = _emb_call(idx, weight)
    return out.reshape(4096, 200, D_MODEL)
